# Initial kernel scaffold; baseline (speedup 1.0000x reference)
import jax
import jax.numpy as jnp
from jax.experimental import pallas as pl


def kernel(prev_outputs, prev_is_active, parent_indices, w, b):
    raise NotImplementedError("write your pallas kernel here")


# speedup vs baseline: 5.7821x; 5.7821x over previous
import functools

import jax
import jax.numpy as jnp
from jax import lax
from jax.experimental import pallas as pl
from jax.experimental.pallas import tpu as pltpu
from jax.experimental.pallas import tpu_sc as plsc

P = 16
THRESH = 12
L = 16
NC = 2
NS = 16


def _sc_body(K, M, D, npw, prev_hbm, flags_hbm, idx_hbm, idxt_hbm, w_hbm,
             b_hbm, out_hbm, act_hbm,
             flags_v, idx_v, idxt_v, w_v, b_v, rows_v, out_v, act_v, wm_v,
             sem0, sem1):
    wid = lax.axis_index("s") * NC + lax.axis_index("c")
    base = wid * npw

    pltpu.sync_copy(flags_hbm, flags_v)
    pltpu.sync_copy(idx_hbm.at[pl.ds(base, npw)], idx_v)
    pltpu.sync_copy(idxt_hbm.at[wid], idxt_v)
    pltpu.sync_copy(w_hbm.at[pl.ds(base, npw)], w_v)
    pltpu.sync_copy(b_hbm.at[pl.ds(base, npw)], b_v)

    def act_chunk(t, _):
        off = t * L
        cnt = jnp.zeros((L,), jnp.int32)
        for p in range(P):
            cnt = cnt + plsc.load_gather(flags_v, [idxt_v[p, pl.ds(off, L)]])
        act_v[pl.ds(off, L)] = (cnt >= THRESH).astype(jnp.int32)
        return 0

    lax.fori_loop(0, npw // L, act_chunk, 0, unroll=False)

    sems = (sem0, sem1)

    def gather_start(i, buf):
        pltpu.make_async_copy(prev_hbm.at[idx_v.at[i]], rows_v.at[buf],
                              sems[buf]).start()

    def gather_wait(i, buf):
        pltpu.make_async_copy(prev_hbm.at[idx_v.at[i]], rows_v.at[buf],
                              sems[buf]).wait()

    def compute_node(i, buf):
        idx_vec = idx_v[i, :]
        flags16 = plsc.load_gather(flags_v, [idx_vec])
        acount = jnp.sum(flags16)
        active = acount >= THRESH
        gate = lax.broadcast(active.astype(jnp.float32), (L,))

        w_row = w_v[i, :]
        wm_v[pl.ds(L, L)] = w_row * flags16.astype(jnp.float32)
        splats = [plsc.load_gather(wm_v, [jnp.full((L,), L + p, jnp.int32)])
                  for p in range(P)]
        b_splat = plsc.load_gather(b_v, [jnp.zeros((L,), jnp.int32) + i])

        gather_wait(i, buf)

        def cbody(c, _):
            off = c * L
            acc = splats[0] * rows_v[buf, 0, pl.ds(off, L)]
            for p in range(1, P):
                acc = acc + splats[p] * rows_v[buf, p, pl.ds(off, L)]
            x = acc + b_splat
            t = jnp.exp(-2.0 * jnp.abs(x))
            y = (1.0 - t) / (1.0 + t)
            y = jnp.where(x < 0.0, -y, y)
            out_v[i, pl.ds(off, L)] = y * gate
            return 0

        lax.fori_loop(0, D // L, cbody, 0, unroll=False)

    gather_start(0, 0)
    gather_start(1, 1)

    def outer(g2, _):
        g = g2 * 2
        for buf in range(2):
            i = g + buf
            compute_node(i, buf)

            @pl.when(i + 2 < npw)
            def _():
                gather_start(i + 2, buf)

        return 0

    lax.fori_loop(0, npw // 2, outer, 0, unroll=False)

    pltpu.sync_copy(out_v, out_hbm.at[pl.ds(base, npw)])
    pltpu.sync_copy(act_v, act_hbm.at[pl.ds(base, npw)])


@functools.partial(jax.jit, static_argnums=(6, 7, 8))
def _decoder_sc(prev_flat, flags_i32, parent_indices, parent_indices_t, w, b,
                K, M, D):
    npw = K // (NC * NS)
    mesh = plsc.VectorSubcoreMesh(core_axis_name="c", subcore_axis_name="s")
    body = functools.partial(_sc_body, K, M, D, npw)
    run = pl.kernel(
        body,
        out_type=[jax.ShapeDtypeStruct((K, D), jnp.float32),
                  jax.ShapeDtypeStruct((K,), jnp.int32)],
        mesh=mesh,
        scratch_types=[
            pltpu.VMEM((M,), jnp.int32),
            pltpu.VMEM((npw, P), jnp.int32),
            pltpu.VMEM((P, npw), jnp.int32),
            pltpu.VMEM((npw, P), jnp.float32),
            pltpu.VMEM((npw,), jnp.float32),
            pltpu.VMEM((2, P, D), jnp.float32),
            pltpu.VMEM((npw, D), jnp.float32),
            pltpu.VMEM((npw,), jnp.int32),
            pltpu.VMEM((2 * P,), jnp.float32),
            pltpu.SemaphoreType.DMA,
            pltpu.SemaphoreType.DMA,
        ],
        compiler_params=pltpu.CompilerParams(needs_layout_passes=False),
    )
    return run(prev_flat, flags_i32, parent_indices, parent_indices_t, w, b)


def kernel(prev_outputs, prev_is_active, parent_indices, w, b):
    M, n, _ = prev_outputs.shape
    K = parent_indices.shape[0]
    D = n * n
    prev_flat = prev_outputs.reshape(M, D)
    flags_i32 = prev_is_active.astype(jnp.int32)
    nw = NC * NS
    idxt3 = parent_indices.reshape(nw, K // nw, P).transpose(0, 2, 1)
    comb, act = _decoder_sc(prev_flat, flags_i32, parent_indices,
                            idxt3, w, b, K, M, D)
    out = comb.reshape(K, 1, n, n)
    return out, act.astype(bool)


# speedup vs baseline: 6.3239x; 1.0937x over previous
import functools

import jax
import jax.numpy as jnp
from jax import lax
from jax.experimental import pallas as pl
from jax.experimental.pallas import tpu as pltpu
from jax.experimental.pallas import tpu_sc as plsc

P = 16
THRESH = 12
L = 16
NC = 2
NS = 16


def _sc_body(K, M, D, npw, prev_hbm, flags_hbm, idx_hbm, idxt_hbm, w_hbm,
             b_hbm, out_hbm, act_hbm,
             flags_v, idx_v, idxt_v, w_v, b_v, rows_v, out_v, act_v, wm_v,
             sem0, sem1):
    wid = lax.axis_index("s") * NC + lax.axis_index("c")
    base = wid * npw

    pltpu.sync_copy(flags_hbm, flags_v)
    pltpu.sync_copy(idx_hbm.at[pl.ds(base, npw)], idx_v)
    pltpu.sync_copy(idxt_hbm.at[wid], idxt_v)
    pltpu.sync_copy(w_hbm.at[pl.ds(base, npw)], w_v)
    pltpu.sync_copy(b_hbm.at[pl.ds(base, npw)], b_v)

    def act_chunk(t, _):
        off = t * L
        cnt = jnp.zeros((L,), jnp.int32)
        for p in range(P):
            cnt = cnt + plsc.load_gather(flags_v, [idxt_v[p, pl.ds(off, L)]])
        act_v[pl.ds(off, L)] = (cnt >= THRESH).astype(jnp.int32)
        return 0

    lax.fori_loop(0, npw // L, act_chunk, 0, unroll=False)

    sems = (sem0, sem1)

    def gather_start(i, buf):
        pltpu.make_async_copy(prev_hbm.at[idx_v.at[i]], rows_v.at[buf],
                              sems[buf]).start()

    def gather_wait(i, buf):
        pltpu.make_async_copy(prev_hbm.at[idx_v.at[i]], rows_v.at[buf],
                              sems[buf]).wait()

    def compute_node(i, buf):
        idx_vec = idx_v[i, :]
        flags16 = plsc.load_gather(flags_v, [idx_vec])
        acount = jnp.sum(flags16)
        active = acount >= THRESH
        gate = lax.broadcast(active.astype(jnp.float32), (L,))

        w_row = w_v[i, :]
        wm_v[pl.ds(L, L)] = w_row * flags16.astype(jnp.float32)
        splats = [plsc.load_gather(wm_v, [jnp.full((L,), L + p, jnp.int32)])
                  for p in range(P)]
        b_splat = plsc.load_gather(b_v, [jnp.zeros((L,), jnp.int32) + i])

        gather_wait(i, buf)

        def cbody(c, _):
            off = c * L
            accs = []
            for q in range(4):
                a = splats[4 * q] * rows_v[buf, 4 * q, pl.ds(off, L)]
                for p in range(4 * q + 1, 4 * q + 4):
                    a = a + splats[p] * rows_v[buf, p, pl.ds(off, L)]
                accs.append(a)
            x = ((accs[0] + accs[1]) + (accs[2] + accs[3])) + b_splat
            t = jnp.exp(-2.0 * jnp.abs(x))
            y = (1.0 - t) / (1.0 + t)
            y = jnp.where(x < 0.0, -y, y)
            out_v[i, pl.ds(off, L)] = y * gate
            return 0

        lax.fori_loop(0, D // L, cbody, 0, unroll=4)

    gather_start(0, 0)
    gather_start(1, 1)

    def outer(g2, _):
        g = g2 * 2
        for buf in range(2):
            i = g + buf
            compute_node(i, buf)

            @pl.when(i + 2 < npw)
            def _():
                gather_start(i + 2, buf)

        return 0

    lax.fori_loop(0, npw // 2, outer, 0, unroll=False)

    pltpu.sync_copy(out_v, out_hbm.at[pl.ds(base, npw)])
    pltpu.sync_copy(act_v, act_hbm.at[pl.ds(base, npw)])


@functools.partial(jax.jit, static_argnums=(6, 7, 8))
def _decoder_sc(prev_flat, flags_i32, parent_indices, parent_indices_t, w, b,
                K, M, D):
    npw = K // (NC * NS)
    mesh = plsc.VectorSubcoreMesh(core_axis_name="c", subcore_axis_name="s")
    body = functools.partial(_sc_body, K, M, D, npw)
    run = pl.kernel(
        body,
        out_type=[jax.ShapeDtypeStruct((K, D), jnp.float32),
                  jax.ShapeDtypeStruct((K,), jnp.int32)],
        mesh=mesh,
        scratch_types=[
            pltpu.VMEM((M,), jnp.int32),
            pltpu.VMEM((npw, P), jnp.int32),
            pltpu.VMEM((P, npw), jnp.int32),
            pltpu.VMEM((npw, P), jnp.float32),
            pltpu.VMEM((npw,), jnp.float32),
            pltpu.VMEM((2, P, D), jnp.float32),
            pltpu.VMEM((npw, D), jnp.float32),
            pltpu.VMEM((npw,), jnp.int32),
            pltpu.VMEM((2 * P,), jnp.float32),
            pltpu.SemaphoreType.DMA,
            pltpu.SemaphoreType.DMA,
        ],
        compiler_params=pltpu.CompilerParams(needs_layout_passes=False),
    )
    return run(prev_flat, flags_i32, parent_indices, parent_indices_t, w, b)


def kernel(prev_outputs, prev_is_active, parent_indices, w, b):
    M, n, _ = prev_outputs.shape
    K = parent_indices.shape[0]
    D = n * n
    prev_flat = prev_outputs.reshape(M, D)
    flags_i32 = prev_is_active.astype(jnp.int32)
    nw = NC * NS
    idxt3 = parent_indices.reshape(nw, K // nw, P).transpose(0, 2, 1)
    comb, act = _decoder_sc(prev_flat, flags_i32, parent_indices,
                            idxt3, w, b, K, M, D)
    out = comb.reshape(K, 1, n, n)
    return out, act.astype(bool)


# speedup vs baseline: 10.1747x; 1.6089x over previous
import functools

import jax
import jax.numpy as jnp
from jax import lax
from jax.experimental import pallas as pl
from jax.experimental.pallas import tpu as pltpu
from jax.experimental.pallas import tpu_sc as plsc

P = 16
THRESH = 12
L = 16
NC = 2
NS = 16


def _sc_body(K, M, D, npw, prev_hbm, flags_hbm, idx_hbm, idxt_hbm, w_hbm,
             b_hbm, out_hbm, act_hbm,
             flags_v, idx_v, idxt_v, w_v, b_v, rows_v, out_v, act_v, wm_v,
             sem0, sem1):
    wid = lax.axis_index("s") * NC + lax.axis_index("c")
    base = wid * npw

    pltpu.sync_copy(flags_hbm, flags_v)
    pltpu.sync_copy(idx_hbm.at[pl.ds(base, npw)], idx_v)
    pltpu.sync_copy(idxt_hbm.at[wid], idxt_v)
    pltpu.sync_copy(w_hbm.at[pl.ds(base, npw)], w_v)
    pltpu.sync_copy(b_hbm.at[pl.ds(base, npw)], b_v)

    def act_chunk(t, _):
        off = t * L
        cnt = jnp.zeros((L,), jnp.int32)
        for p in range(P):
            cnt = cnt + plsc.load_gather(flags_v, [idxt_v[p, pl.ds(off, L)]])
        act_v[pl.ds(off, L)] = (cnt >= THRESH).astype(jnp.int32)
        return 0

    lax.fori_loop(0, npw // L, act_chunk, 0, unroll=False)

    sems = (sem0, sem1)

    def gather_start(i, buf):
        pltpu.make_async_copy(prev_hbm.at[idx_v.at[i]], rows_v.at[buf],
                              sems[buf]).start()

    def gather_wait(i, buf):
        pltpu.make_async_copy(prev_hbm.at[idx_v.at[i]], rows_v.at[buf],
                              sems[buf]).wait()

    def compute_node(i, buf):
        idx_vec = idx_v[i, :]
        flags16 = plsc.load_gather(flags_v, [idx_vec])
        acount = jnp.sum(flags16)
        active = acount >= THRESH
        gate = lax.broadcast(active.astype(jnp.float32), (L,))

        w_row = w_v[i, :]
        wm_v[pl.ds(L, L)] = w_row * flags16.astype(jnp.float32) * gate
        splats = [plsc.load_gather(wm_v, [jnp.full((L,), L + p, jnp.int32)])
                  for p in range(P)]
        b_splat = plsc.load_gather(b_v, [jnp.zeros((L,), jnp.int32) + i]) * gate

        gather_wait(i, buf)

        @plsc.parallel_loop(0, D // L, unroll=4)
        def _(c):
            off = c * L
            accs = []
            for q in range(4):
                a = splats[4 * q] * rows_v[buf, 4 * q, pl.ds(off, L)]
                for p in range(4 * q + 1, 4 * q + 4):
                    a = a + splats[p] * rows_v[buf, p, pl.ds(off, L)]
                accs.append(a)
            x = ((accs[0] + accs[1]) + (accs[2] + accs[3])) + b_splat
            xi = plsc.bitcast(x, jnp.uint32)
            sign = xi & jnp.uint32(0x80000000)
            ax = plsc.bitcast(xi & jnp.uint32(0x7FFFFFFF), jnp.float32)
            t = jnp.exp(-2.0 * ax)
            y = (1.0 - t) / (1.0 + t)
            yi = plsc.bitcast(y, jnp.uint32) | sign
            out_v[i, pl.ds(off, L)] = plsc.bitcast(yi, jnp.float32)

    gather_start(0, 0)
    gather_start(1, 1)

    def outer(g2, _):
        g = g2 * 2
        for buf in range(2):
            i = g + buf
            compute_node(i, buf)

            @pl.when(i + 2 < npw)
            def _():
                gather_start(i + 2, buf)

        return 0

    lax.fori_loop(0, npw // 2, outer, 0, unroll=False)

    pltpu.sync_copy(out_v, out_hbm.at[pl.ds(base, npw)])
    pltpu.sync_copy(act_v, act_hbm.at[pl.ds(base, npw)])


@functools.partial(jax.jit, static_argnums=(6, 7, 8))
def _decoder_sc(prev_flat, flags_i32, parent_indices, parent_indices_t, w, b,
                K, M, D):
    npw = K // (NC * NS)
    mesh = plsc.VectorSubcoreMesh(core_axis_name="c", subcore_axis_name="s")
    body = functools.partial(_sc_body, K, M, D, npw)
    run = pl.kernel(
        body,
        out_type=[jax.ShapeDtypeStruct((K, D), jnp.float32),
                  jax.ShapeDtypeStruct((K,), jnp.int32)],
        mesh=mesh,
        scratch_types=[
            pltpu.VMEM((M,), jnp.int32),
            pltpu.VMEM((npw, P), jnp.int32),
            pltpu.VMEM((P, npw), jnp.int32),
            pltpu.VMEM((npw, P), jnp.float32),
            pltpu.VMEM((npw,), jnp.float32),
            pltpu.VMEM((2, P, D), jnp.float32),
            pltpu.VMEM((npw, D), jnp.float32),
            pltpu.VMEM((npw,), jnp.int32),
            pltpu.VMEM((2 * P,), jnp.float32),
            pltpu.SemaphoreType.DMA,
            pltpu.SemaphoreType.DMA,
        ],
        compiler_params=pltpu.CompilerParams(needs_layout_passes=False),
    )
    return run(prev_flat, flags_i32, parent_indices, parent_indices_t, w, b)


def kernel(prev_outputs, prev_is_active, parent_indices, w, b):
    M, n, _ = prev_outputs.shape
    K = parent_indices.shape[0]
    D = n * n
    prev_flat = prev_outputs.reshape(M, D)
    flags_i32 = prev_is_active.astype(jnp.int32)
    nw = NC * NS
    idxt3 = parent_indices.reshape(nw, K // nw, P).transpose(0, 2, 1)
    comb, act = _decoder_sc(prev_flat, flags_i32, parent_indices,
                            idxt3, w, b, K, M, D)
    out = comb.reshape(K, 1, n, n)
    return out, act.astype(bool)
